# Initial kernel scaffold; baseline (speedup 1.0000x reference)
import jax
import jax.numpy as jnp
from jax.experimental import pallas as pl


def kernel(x, edge_index, W_in, b_in, W_out, b_out, W_env, b_env, ln_in_g, ln_in_b, ln_out_g, ln_out_b):
    raise NotImplementedError("write your pallas kernel here")


# speedup vs baseline: 23.3766x; 23.3766x over previous
import functools

import jax
import jax.numpy as jnp
from jax import lax
from jax.experimental import pallas as pl
from jax.experimental.pallas import tpu as pltpu
from jax.experimental.pallas import tpu_sc as plsc

N = 10000
E = 320000
D = 128
TEMP = 0.5

NC, NS, L = 2, 16, 16
NW = NC * NS
NB = 79
E_PAD = NW * NB * 128
N_ACC = 10112
RPT = N_ACC // NS


def _make_scatter(Dw):
    mesh = plsc.VectorSubcoreMesh(core_axis_name="c", subcore_axis_name="s")

    @functools.partial(
        pl.kernel,
        mesh=mesh,
        compiler_params=pltpu.CompilerParams(use_tc_tiling_on_sc=False),
        out_type=jax.ShapeDtypeStruct((NC, N_ACC, Dw), jnp.float32),
        scratch_types=[
            pltpu.VMEM((NB, 128), jnp.int32),
            pltpu.VMEM((NB, 128), jnp.int32),
            pltpu.VMEM((128, Dw), jnp.float32),
            pltpu.VMEM_SHARED((N_ACC, Dw), jnp.float32),
            pltpu.SemaphoreType.DMA,
        ],
    )
    def k(src_hbm, dst_hbm, table_hbm, zrow_hbm, out_hbm,
          src_v, dst_v, rows_v, acc, sem):
        cid = lax.axis_index("c")
        sid = lax.axis_index("s")
        wid = sid * NC + cid
        pltpu.sync_copy(zrow_hbm, acc.at[pl.ds(sid * RPT, RPT)])
        pltpu.sync_copy(src_hbm.at[wid], src_v)
        pltpu.sync_copy(dst_hbm.at[wid], dst_v)
        plsc.subcore_barrier()

        def body(j, carry):
            pltpu.async_copy(table_hbm.at[src_v.at[j]], rows_v, sem).wait()
            pltpu.sync_copy(rows_v, acc.at[dst_v.at[j]], add=True)
            return carry

        lax.fori_loop(0, NB, body, 0)
        plsc.subcore_barrier()
        pltpu.sync_copy(acc.at[pl.ds(sid * RPT, RPT)],
                        out_hbm.at[cid, pl.ds(sid * RPT, RPT)])

    return k


_scatter8 = _make_scatter(8)
_scatter128 = _make_scatter(128)


def _front_body(x_ref, g_ref, b_ref, w4_ref, wenv_ref, h4_ref, henv_ref):
    x = x_ref[...]
    mu = jnp.mean(x, axis=-1, keepdims=True)
    var = jnp.mean((x - mu) ** 2, axis=-1, keepdims=True)
    xn = (x - mu) / jnp.sqrt(var + 1e-5) * g_ref[...] + b_ref[...]
    h4_ref[...] = jnp.dot(xn, w4_ref[...], preferred_element_type=jnp.float32)
    henv_ref[...] = jnp.dot(xn, wenv_ref[...], preferred_element_type=jnp.float32)


def _front(x, ln_g, ln_b, W4, W_env, bs=2000):
    grid = (N // bs,)
    return pl.pallas_call(
        _front_body,
        grid=grid,
        in_specs=[
            pl.BlockSpec((bs, D), lambda i: (i, 0)),
            pl.BlockSpec((1, D), lambda i: (0, 0)),
            pl.BlockSpec((1, D), lambda i: (0, 0)),
            pl.BlockSpec((D, 4), lambda i: (0, 0)),
            pl.BlockSpec((D, D), lambda i: (0, 0)),
        ],
        out_specs=[
            pl.BlockSpec((bs, 4), lambda i: (i, 0)),
            pl.BlockSpec((bs, D), lambda i: (i, 0)),
        ],
        out_shape=[
            jax.ShapeDtypeStruct((N, 4), jnp.float32),
            jax.ShapeDtypeStruct((N, D), jnp.float32),
        ],
    )(x, ln_g.reshape(1, D), ln_b.reshape(1, D), W4, W_env)


def _scale_body(a_ref, h_ref, o_ref):
    o_ref[...] = a_ref[...] * h_ref[...]


def _scale_rows(a, h, bs=2000):
    return pl.pallas_call(
        _scale_body,
        grid=(N // bs,),
        in_specs=[
            pl.BlockSpec((bs, 1), lambda i: (i, 0)),
            pl.BlockSpec((bs, D), lambda i: (i, 0)),
        ],
        out_specs=pl.BlockSpec((bs, D), lambda i: (i, 0)),
        out_shape=jax.ShapeDtypeStruct((N, D), jnp.float32),
    )(a.reshape(N, 1), h)


def _final_body(p0_ref, p1_ref, henv_ref, c1_ref, c2_ref, be_ref,
                g_ref, b_ref, o_ref):
    o = (c1_ref[...] * (p0_ref[...] + p1_ref[...])
         + c2_ref[...] * henv_ref[...] + be_ref[...])
    mu = jnp.mean(o, axis=-1, keepdims=True)
    var = jnp.mean((o - mu) ** 2, axis=-1, keepdims=True)
    o_ref[...] = (o - mu) / jnp.sqrt(var + 1e-5) * g_ref[...] + b_ref[...]


def _final(p0, p1, h_env, c1, c2, b_env, ln_g, ln_b, bs=2000):
    return pl.pallas_call(
        _final_body,
        grid=(N // bs,),
        in_specs=[
            pl.BlockSpec((bs, D), lambda i: (i, 0)),
            pl.BlockSpec((bs, D), lambda i: (i, 0)),
            pl.BlockSpec((bs, D), lambda i: (i, 0)),
            pl.BlockSpec((bs, 1), lambda i: (i, 0)),
            pl.BlockSpec((bs, 1), lambda i: (i, 0)),
            pl.BlockSpec((1, D), lambda i: (0, 0)),
            pl.BlockSpec((1, D), lambda i: (0, 0)),
            pl.BlockSpec((1, D), lambda i: (0, 0)),
        ],
        out_specs=pl.BlockSpec((bs, D), lambda i: (i, 0)),
        out_shape=jax.ShapeDtypeStruct((N, D), jnp.float32),
    )(p0, p1, h_env, c1.reshape(N, 1), c2.reshape(N, 1),
      b_env.reshape(1, D), ln_g.reshape(1, D), ln_b.reshape(1, D))


def _gumbel_hard0(logits, g):
    y = jax.nn.softmax((logits + g) / TEMP, axis=-1)
    idx = jnp.argmax(y, axis=-1)
    y_hard = jax.nn.one_hot(idx, 2, dtype=y.dtype)
    return ((y_hard - y) + y)[:, 0]


def _pad_table(t):
    return jnp.concatenate(
        [t, jnp.zeros((N_ACC - N, t.shape[1]), jnp.float32)], axis=0)


def kernel(x, edge_index, W_in, b_in, W_out, b_out, W_env, b_env,
           ln_in_g, ln_in_b, ln_out_g, ln_out_b):
    src, dst = edge_index[0], edge_index[1]
    pad = jnp.full((E_PAD - E,), N, dtype=jnp.int32)
    src3 = jnp.concatenate([src, pad]).reshape(NW, NB, 128)
    dst3 = jnp.concatenate([dst, pad]).reshape(NW, NB, 128)

    W4 = jnp.concatenate([W_in, W_out], axis=1)
    b4 = jnp.concatenate([b_in, b_out])
    h4, h_env = _front(x, ln_in_g, ln_in_b, W4, W_env)

    zrow8 = jnp.zeros((RPT, 8), jnp.float32)
    zrow128 = jnp.zeros((RPT, D), jnp.float32)

    def to8(t):
        return jnp.concatenate(
            [t, jnp.zeros((N, 8 - t.shape[1]), jnp.float32)], axis=1)

    ones_t = _pad_table(to8(jnp.ones((N, 1), jnp.float32)))
    cnt = _scatter8(src3, dst3, ones_t, zrow8)
    cnt = cnt[0, :N, 0] + cnt[1, :N, 0]
    dinv_u = 1.0 / jnp.sqrt(cnt + 1.0)

    h4s = _pad_table(to8(dinv_u[:, None] * h4))
    pre4 = _scatter8(src3, dst3, h4s, zrow8)
    pre4 = pre4[0, :N, :4] + pre4[1, :N, :4]
    logits4 = dinv_u[:, None] * pre4 + (dinv_u ** 2)[:, None] * h4 + b4

    kg = jax.random.key(42)
    u1 = jax.random.uniform(jax.random.fold_in(kg, 0), (N, 2),
                            minval=1e-6, maxval=1.0)
    u2 = jax.random.uniform(jax.random.fold_in(kg, 1), (N, 2),
                            minval=1e-6, maxval=1.0)
    g1 = -jnp.log(-jnp.log(u1))
    g2 = -jnp.log(-jnp.log(u2))
    in_val = _gumbel_hard0(logits4[:, :2], g1)
    out_val = _gumbel_hard0(logits4[:, 2:], g2)

    s_out = _scatter8(src3, dst3, _pad_table(to8(out_val[:, None])), zrow8)
    s_out = s_out[0, :N, 0] + s_out[1, :N, 0]
    deg_w = in_val * s_out + 1.0
    dinv_w = 1.0 / jnp.sqrt(deg_w)

    hh = _pad_table(_scale_rows(out_val * dinv_w, h_env))
    pre = _scatter128(src3, dst3, hh, zrow128)

    c1 = dinv_w * in_val
    c2 = dinv_w ** 2
    return _final(pre[0, :N], pre[1, :N], h_env, c1, c2,
                  b_env, ln_out_g, ln_out_b)


# speedup vs baseline: 26.0990x; 1.1165x over previous
import functools

import jax
import jax.numpy as jnp
from jax import lax
from jax.experimental import pallas as pl
from jax.experimental.pallas import tpu as pltpu
from jax.experimental.pallas import tpu_sc as plsc

N = 10000
E = 320000
D = 128
TEMP = 0.5

NC, NS, L = 2, 16, 16
NW = NC * NS
NB = 80
E_PAD = NW * NB * 128
N_ACC = 10112
RPT = N_ACC // NS


def _make_scatter(Dw, C):
    NBC = NB // C
    mesh = plsc.VectorSubcoreMesh(core_axis_name="c", subcore_axis_name="s")

    @functools.partial(
        pl.kernel,
        mesh=mesh,
        compiler_params=pltpu.CompilerParams(use_tc_tiling_on_sc=False),
        out_type=jax.ShapeDtypeStruct((NC, N_ACC, Dw), jnp.float32),
        scratch_types=[
            pltpu.VMEM((NB, 128), jnp.int32),
            pltpu.VMEM((NB, 128), jnp.int32),
            pltpu.VMEM((C, 128, Dw), jnp.float32),
            pltpu.VMEM_SHARED((N_ACC, Dw), jnp.float32),
            pltpu.SemaphoreType.DMA,
            pltpu.SemaphoreType.DMA,
        ],
    )
    def k(src_hbm, dst_hbm, table_hbm, zrow_hbm, out_hbm,
          src_v, dst_v, rows_v, acc, sem_g, sem_s):
        cid = lax.axis_index("c")
        sid = lax.axis_index("s")
        wid = sid * NC + cid
        pltpu.sync_copy(zrow_hbm, acc.at[pl.ds(sid * RPT, RPT)])
        pltpu.sync_copy(src_hbm.at[wid], src_v)
        pltpu.sync_copy(dst_hbm.at[wid], dst_v)
        plsc.subcore_barrier()

        def chunk(i, carry):
            gh = [pltpu.async_copy(table_hbm.at[src_v.at[i * C + b]],
                                   rows_v.at[b], sem_g)
                  for b in range(C)]
            sh = []
            for b in range(C):
                gh[b].wait()
                sh.append(pltpu.async_copy(rows_v.at[b],
                                           acc.at[dst_v.at[i * C + b]],
                                           sem_s, add=True))
            for b in range(C):
                sh[b].wait()
            return carry

        lax.fori_loop(0, NBC, chunk, 0)
        plsc.subcore_barrier()
        pltpu.sync_copy(acc.at[pl.ds(sid * RPT, RPT)],
                        out_hbm.at[cid, pl.ds(sid * RPT, RPT)])

    return k


_scatter8 = _make_scatter(8, 8)

NBT = E_PAD // (NS * 128)
CS = 5
_mesh_split = plsc.VectorSubcoreMesh(core_axis_name="c", subcore_axis_name="s")


@functools.partial(
    pl.kernel,
    mesh=_mesh_split,
    compiler_params=pltpu.CompilerParams(use_tc_tiling_on_sc=False),
    out_type=jax.ShapeDtypeStruct((NC, N_ACC, 64), jnp.float32),
    scratch_types=[
        pltpu.VMEM((NBT, 128), jnp.int32),
        pltpu.VMEM((NBT, 128), jnp.int32),
        pltpu.VMEM((CS, 128, 64), jnp.float32),
        pltpu.VMEM_SHARED((N_ACC, 64), jnp.float32),
        pltpu.SemaphoreType.DMA,
        pltpu.SemaphoreType.DMA,
    ],
)
def _scatter_split(src_hbm, dst_hbm, table_hbm, zrow_hbm, out_hbm,
                   src_v, dst_v, rows_v, acc, sem_g, sem_s):
    cid = lax.axis_index("c")
    sid = lax.axis_index("s")
    pltpu.sync_copy(zrow_hbm, acc.at[pl.ds(sid * RPT, RPT)])
    pltpu.sync_copy(src_hbm.at[sid], src_v)
    pltpu.sync_copy(dst_hbm.at[sid], dst_v)
    plsc.subcore_barrier()

    def chunk(i, carry):
        gh = [pltpu.async_copy(table_hbm.at[cid].at[src_v.at[i * CS + b]],
                               rows_v.at[b], sem_g)
              for b in range(CS)]
        sh = []
        for b in range(CS):
            gh[b].wait()
            sh.append(pltpu.async_copy(rows_v.at[b],
                                       acc.at[dst_v.at[i * CS + b]],
                                       sem_s, add=True))
        for b in range(CS):
            sh[b].wait()
        return carry

    lax.fori_loop(0, NBT // CS, chunk, 0)
    plsc.subcore_barrier()
    pltpu.sync_copy(acc.at[pl.ds(sid * RPT, RPT)],
                    out_hbm.at[cid, pl.ds(sid * RPT, RPT)])


def _front_body(x_ref, g_ref, b_ref, w4_ref, wenv_ref, h4_ref, henv_ref):
    x = x_ref[...]
    mu = jnp.mean(x, axis=-1, keepdims=True)
    var = jnp.mean((x - mu) ** 2, axis=-1, keepdims=True)
    xn = (x - mu) / jnp.sqrt(var + 1e-5) * g_ref[...] + b_ref[...]
    h4_ref[...] = jnp.dot(xn, w4_ref[...], preferred_element_type=jnp.float32)
    henv_ref[...] = jnp.dot(xn, wenv_ref[...], preferred_element_type=jnp.float32)


def _front(x, ln_g, ln_b, W4, W_env, bs=2000):
    grid = (N // bs,)
    return pl.pallas_call(
        _front_body,
        grid=grid,
        in_specs=[
            pl.BlockSpec((bs, D), lambda i: (i, 0)),
            pl.BlockSpec((1, D), lambda i: (0, 0)),
            pl.BlockSpec((1, D), lambda i: (0, 0)),
            pl.BlockSpec((D, 4), lambda i: (0, 0)),
            pl.BlockSpec((D, D), lambda i: (0, 0)),
        ],
        out_specs=[
            pl.BlockSpec((bs, 4), lambda i: (i, 0)),
            pl.BlockSpec((bs, D), lambda i: (i, 0)),
        ],
        out_shape=[
            jax.ShapeDtypeStruct((N, 4), jnp.float32),
            jax.ShapeDtypeStruct((N, D), jnp.float32),
        ],
    )(x, ln_g.reshape(1, D), ln_b.reshape(1, D), W4, W_env)


def _scale_body(a_ref, h_ref, o_ref):
    hh = a_ref[...] * h_ref[...]
    o_ref[0] = hh[:, :64]
    o_ref[1] = hh[:, 64:]


def _scale_rows_split(a, h, bs=2000):
    return pl.pallas_call(
        _scale_body,
        grid=(N // bs,),
        in_specs=[
            pl.BlockSpec((bs, 1), lambda i: (i, 0)),
            pl.BlockSpec((bs, D), lambda i: (i, 0)),
        ],
        out_specs=pl.BlockSpec((NC, bs, 64), lambda i: (0, i, 0)),
        out_shape=jax.ShapeDtypeStruct((NC, N, 64), jnp.float32),
    )(a.reshape(N, 1), h)


def _final_body(p0_ref, p1_ref, henv_ref, c1_ref, c2_ref, be_ref,
                g_ref, b_ref, o_ref):
    pre = jnp.concatenate([p0_ref[...], p1_ref[...]], axis=1)
    o = (c1_ref[...] * pre
         + c2_ref[...] * henv_ref[...] + be_ref[...])
    mu = jnp.mean(o, axis=-1, keepdims=True)
    var = jnp.mean((o - mu) ** 2, axis=-1, keepdims=True)
    o_ref[...] = (o - mu) / jnp.sqrt(var + 1e-5) * g_ref[...] + b_ref[...]


def _final(p0, p1, h_env, c1, c2, b_env, ln_g, ln_b, bs=2000):
    return pl.pallas_call(
        _final_body,
        grid=(N // bs,),
        in_specs=[
            pl.BlockSpec((bs, 64), lambda i: (i, 0)),
            pl.BlockSpec((bs, 64), lambda i: (i, 0)),
            pl.BlockSpec((bs, D), lambda i: (i, 0)),
            pl.BlockSpec((bs, 1), lambda i: (i, 0)),
            pl.BlockSpec((bs, 1), lambda i: (i, 0)),
            pl.BlockSpec((1, D), lambda i: (0, 0)),
            pl.BlockSpec((1, D), lambda i: (0, 0)),
            pl.BlockSpec((1, D), lambda i: (0, 0)),
        ],
        out_specs=pl.BlockSpec((bs, D), lambda i: (i, 0)),
        out_shape=jax.ShapeDtypeStruct((N, D), jnp.float32),
    )(p0, p1, h_env, c1.reshape(N, 1), c2.reshape(N, 1),
      b_env.reshape(1, D), ln_g.reshape(1, D), ln_b.reshape(1, D))


def _gumbel_hard0(logits, g):
    y = jax.nn.softmax((logits + g) / TEMP, axis=-1)
    idx = jnp.argmax(y, axis=-1)
    y_hard = jax.nn.one_hot(idx, 2, dtype=y.dtype)
    return ((y_hard - y) + y)[:, 0]


def _pad_table(t):
    return jnp.concatenate(
        [t, jnp.zeros((N_ACC - N, t.shape[1]), jnp.float32)], axis=0)


def kernel(x, edge_index, W_in, b_in, W_out, b_out, W_env, b_env,
           ln_in_g, ln_in_b, ln_out_g, ln_out_b):
    src, dst = edge_index[0], edge_index[1]
    pad = jnp.full((E_PAD - E,), N, dtype=jnp.int32)
    src3 = jnp.concatenate([src, pad]).reshape(NW, NB, 128)
    dst3 = jnp.concatenate([dst, pad]).reshape(NW, NB, 128)

    W4 = jnp.concatenate([W_in, W_out], axis=1)
    b4 = jnp.concatenate([b_in, b_out])
    h4, h_env = _front(x, ln_in_g, ln_in_b, W4, W_env)

    zrow8 = jnp.zeros((RPT, 8), jnp.float32)

    def to8(t):
        return jnp.concatenate(
            [t, jnp.zeros((N, 8 - t.shape[1]), jnp.float32)], axis=1)

    ones_t = _pad_table(to8(jnp.ones((N, 1), jnp.float32)))
    cnt = _scatter8(src3, dst3, ones_t, zrow8)
    cnt = cnt[0, :N, 0] + cnt[1, :N, 0]
    dinv_u = 1.0 / jnp.sqrt(cnt + 1.0)

    h4s = _pad_table(to8(dinv_u[:, None] * h4))
    pre4 = _scatter8(src3, dst3, h4s, zrow8)
    pre4 = pre4[0, :N, :4] + pre4[1, :N, :4]
    logits4 = dinv_u[:, None] * pre4 + (dinv_u ** 2)[:, None] * h4 + b4

    kg = jax.random.key(42)
    u1 = jax.random.uniform(jax.random.fold_in(kg, 0), (N, 2),
                            minval=1e-6, maxval=1.0)
    u2 = jax.random.uniform(jax.random.fold_in(kg, 1), (N, 2),
                            minval=1e-6, maxval=1.0)
    g1 = -jnp.log(-jnp.log(u1))
    g2 = -jnp.log(-jnp.log(u2))
    in_val = _gumbel_hard0(logits4[:, :2], g1)
    out_val = _gumbel_hard0(logits4[:, 2:], g2)

    s_out = _scatter8(src3, dst3, _pad_table(to8(out_val[:, None])), zrow8)
    s_out = s_out[0, :N, 0] + s_out[1, :N, 0]
    deg_w = in_val * s_out + 1.0
    dinv_w = 1.0 / jnp.sqrt(deg_w)

    hh = _scale_rows_split(out_val * dinv_w, h_env)
    hh = jnp.concatenate(
        [hh, jnp.zeros((NC, N_ACC - N, 64), jnp.float32)], axis=1)
    src3s = jnp.concatenate([src, pad]).reshape(NS, NBT, 128)
    dst3s = jnp.concatenate([dst, pad]).reshape(NS, NBT, 128)
    zrow64 = jnp.zeros((RPT, 64), jnp.float32)
    pre = _scatter_split(src3s, dst3s, hh, zrow64)

    c1 = dinv_w * in_val
    c2 = dinv_w ** 2
    return _final(pre[0, :N], pre[1, :N], h_env, c1, c2,
                  b_env, ln_out_g, ln_out_b)
